# Initial kernel scaffold; baseline (speedup 1.0000x reference)
import jax
import jax.numpy as jnp
from jax.experimental import pallas as pl


def kernel(features, edge_index, edge_weight, fouriercoeffs, bias):
    raise NotImplementedError("write your pallas kernel here")


# speedup vs baseline: 4.0878x; 4.0878x over previous
import functools
import jax
import jax.numpy as jnp
from jax import lax
from jax.experimental import pallas as pl
from jax.experimental.pallas import tpu as pltpu
from jax.experimental.pallas import tpu_sc as plsc

N_NODES = 10000
FEAT = 128
N_EDGES = 320000
GRID_K = 5

NC = 2
NS = 16
NW = NC * NS
E_PER_W = N_EDGES // NW
EB = 80
NB = E_PER_W // EB
NPAD = 10240
ROWS_PER_TILE = NPAD // NS
FCH = FEAT // 16


def _spmv_body(feat_hbm, src_hbm, dst_hbm, ew_hbm, out_hbm, rows_v, src_v,
               dst_v, w_v, shared, sem):
    c = lax.axis_index("c")
    s = lax.axis_index("s")
    wid = s * NC + c

    def zero_row(r, carry):
        for ch in range(FCH):
            rows_v[r, pl.ds(ch * 16, 16)] = jnp.zeros((16,), jnp.float32)
        return carry

    lax.fori_loop(0, EB, zero_row, 0)
    stripe = s * ROWS_PER_TILE
    for j in range(ROWS_PER_TILE // EB):
        pltpu.sync_copy(rows_v, shared.at[pl.ds(stripe + j * EB, EB)])
    plsc.subcore_barrier()

    def batch(b, carry):
        base = wid * E_PER_W + b * EB
        pltpu.sync_copy(src_hbm.at[pl.ds(base, EB)], src_v)
        pltpu.sync_copy(dst_hbm.at[pl.ds(base, EB)], dst_v)
        pltpu.sync_copy(ew_hbm.at[pl.ds(base, EB)], w_v)
        pltpu.async_copy(feat_hbm.at[src_v], rows_v, sem).wait()

        def scale_grp(g, inner):
            r0 = g * 16
            w16 = w_v[pl.ds(r0, 16)]
            for l in range(16):
                w = w16[l]
                for ch in range(FCH):
                    rows_v[r0 + l, pl.ds(ch * 16, 16)] = (
                        rows_v[r0 + l, pl.ds(ch * 16, 16)] * w)
            return inner

        lax.fori_loop(0, EB // 16, scale_grp, 0)
        pltpu.sync_copy(rows_v, shared.at[dst_v], add=True)
        return carry

    lax.fori_loop(0, NB, batch, 0)
    plsc.subcore_barrier()

    for j in range(ROWS_PER_TILE // EB):
        off = stripe + j * EB
        pltpu.sync_copy(shared.at[pl.ds(off, EB)], out_hbm.at[c, pl.ds(off, EB)])


@functools.partial(jax.jit, static_argnames=())
def _spmv(features, edge_src, edge_dst, edge_weight):
    mesh = plsc.VectorSubcoreMesh(core_axis_name="c", subcore_axis_name="s")
    f = pl.kernel(
        _spmv_body,
        out_type=jax.ShapeDtypeStruct((NC, NPAD, FEAT), jnp.float32),
        mesh=mesh,
        scratch_types=[
            pltpu.VMEM((EB, FEAT), jnp.float32),
            pltpu.VMEM((EB,), jnp.int32),
            pltpu.VMEM((EB,), jnp.int32),
            pltpu.VMEM((EB,), jnp.float32),
            pltpu.VMEM_SHARED((NPAD, FEAT), jnp.float32),
            pltpu.SemaphoreType.DMA,
        ],
    )
    return f(features, edge_src, edge_dst, edge_weight)


BLK = 256
NBLK = (N_NODES + BLK - 1) // BLK


def _dense_body(feat_ref, xp_ref, c0_ref, c1_ref, bias_ref, out_ref):
    f = feat_ref[...]
    x = xp_ref[0] + xp_ref[1]
    inter = x * f
    cs = jnp.concatenate([jnp.cos(inter * float(g)) for g in range(1, GRID_K + 1)],
                         axis=1)
    sn = jnp.concatenate([jnp.sin(inter * float(g)) for g in range(1, GRID_K + 1)],
                         axis=1)
    y = (jnp.dot(cs, c0_ref[...], preferred_element_type=jnp.float32)
         + jnp.dot(sn, c1_ref[...], preferred_element_type=jnp.float32)
         + bias_ref[...])
    out_ref[...] = f + x + y


def _dense(features, xp, c0, c1, bias):
    return pl.pallas_call(
        _dense_body,
        grid=(NBLK,),
        in_specs=[
            pl.BlockSpec((BLK, FEAT), lambda i: (i, 0)),
            pl.BlockSpec((NC, BLK, FEAT), lambda i: (0, i, 0)),
            pl.BlockSpec((FEAT * GRID_K, FEAT), lambda i: (0, 0)),
            pl.BlockSpec((FEAT * GRID_K, FEAT), lambda i: (0, 0)),
            pl.BlockSpec((1, FEAT), lambda i: (0, 0)),
        ],
        out_specs=pl.BlockSpec((BLK, FEAT), lambda i: (i, 0)),
        out_shape=jax.ShapeDtypeStruct((N_NODES, FEAT), jnp.float32),
    )(features, xp, c0, c1, bias)


def kernel(features, edge_index, edge_weight, fouriercoeffs, bias):
    xp = _spmv(features, edge_index[0], edge_index[1], edge_weight)
    c0 = fouriercoeffs[0].transpose(2, 1, 0).reshape(FEAT * GRID_K, FEAT)
    c1 = fouriercoeffs[1].transpose(2, 1, 0).reshape(FEAT * GRID_K, FEAT)
    return _dense(features, xp, c0, c1, bias)
